# Initial kernel scaffold; baseline (speedup 1.0000x reference)
import jax
import jax.numpy as jnp
from jax.experimental import pallas as pl


def kernel(region_features, region_edges, region_features_batch, W1, b1, W2, b2, W3, b3):
    raise NotImplementedError("write your pallas kernel here")


# speedup vs baseline: 14.6699x; 14.6699x over previous
import functools

import jax
import jax.numpy as jnp
from jax import lax
from jax.experimental import pallas as pl
from jax.experimental.pallas import tpu as pltpu
from jax.experimental.pallas import tpu_sc as plsc

N = 10000
E = 320000
D = 128
G = 64
NC = 2
NS = 16
NW = NC * NS
CHUNK = 128
GRP = 8
RPS = 624
REM = N - NS * RPS
NDUMP = 8
NP = N + NDUMP
BLK = 1000
GRID = N // BLK

_sc_mesh2 = plsc.VectorSubcoreMesh(core_axis_name="c", subcore_axis_name="s")
_sc_mesh1 = plsc.VectorSubcoreMesh(core_axis_name="c", subcore_axis_name="s",
                                   num_cores=1)

DEG_EPW = E // NW
AGG_NCH = 160
AGG_NGRP = AGG_NCH // GRP
EPAD = NS * AGG_NCH * CHUNK - E


@functools.partial(
    pl.kernel,
    mesh=_sc_mesh2,
    out_type=jax.ShapeDtypeStruct((NW, N), jnp.float32),
    scratch_types=[
        pltpu.VMEM((N,), jnp.float32),
        pltpu.VMEM((DEG_EPW,), jnp.int32),
    ],
    compiler_params=pltpu.CompilerParams(needs_layout_passes=False),
)
def _deg_kernel(dst_hbm, out_hbm, hist, dst_v):
    c = lax.axis_index("c")
    s = lax.axis_index("s")
    wid = s * NC + c

    zero16 = jnp.zeros((16,), jnp.float32)

    def zbody(i, carry):
        hist[pl.ds(i * 16, 16)] = zero16
        return carry

    lax.fori_loop(0, N // 16, zbody, 0)

    pltpu.sync_copy(dst_hbm.at[pl.ds(wid * DEG_EPW, DEG_EPW)], dst_v)

    ones16 = jnp.ones((16,), jnp.float32)

    def body(i, carry):
        idx = dst_v[pl.ds(i * 16, 16)]
        plsc.addupdate_scatter(hist, [idx], ones16)
        return carry

    lax.fori_loop(0, DEG_EPW // 16, body, 0)

    pltpu.sync_copy(hist, out_hbm.at[wid])


@functools.partial(
    pl.kernel,
    mesh=_sc_mesh1,
    out_type=jax.ShapeDtypeStruct((NP, D), jnp.float32),
    scratch_types=[
        pltpu.VMEM((2, GRP, CHUNK), jnp.int32),
        pltpu.VMEM((2, GRP, CHUNK), jnp.int32),
        pltpu.VMEM((2, CHUNK, D), jnp.float32),
        pltpu.VMEM_SHARED((NP, D), jnp.float32),
        pltpu.SemaphoreType.DMA,
        pltpu.SemaphoreType.DMA,
        pltpu.SemaphoreType.DMA,
    ],
    compiler_params=pltpu.CompilerParams(needs_layout_passes=False),
)
def _agg_kernel(y_hbm, src_hbm, dst_hbm, out_hbm,
                src_v, dst_v, rows, acc, gsem, ssem, isem):
    s = lax.axis_index("s")

    row0 = s * RPS
    pltpu.sync_copy(y_hbm.at[pl.ds(row0, RPS)], acc.at[pl.ds(row0, RPS)])

    @pl.when(s == NS - 1)
    def _():
        pltpu.sync_copy(y_hbm.at[pl.ds(NS * RPS, REM)],
                        acc.at[pl.ds(NS * RPS, REM)])

    base0 = s * AGG_NCH
    pltpu.async_copy(src_hbm.at[pl.ds(base0, GRP)], src_v.at[0], isem)
    pltpu.async_copy(dst_hbm.at[pl.ds(base0, GRP)], dst_v.at[0], isem)

    plsc.subcore_barrier()

    def group(g, carry):
        par = g % 2
        base = base0 + g * GRP
        pltpu.make_async_copy(src_hbm.at[pl.ds(base, GRP)],
                              src_v.at[par], isem).wait()
        pltpu.make_async_copy(dst_hbm.at[pl.ds(base, GRP)],
                              dst_v.at[par], isem).wait()

        @pl.when(g + 1 < AGG_NGRP)
        def _():
            nbase = base + GRP
            pltpu.async_copy(src_hbm.at[pl.ds(nbase, GRP)],
                             src_v.at[1 - par], isem)
            pltpu.async_copy(dst_hbm.at[pl.ds(nbase, GRP)],
                             dst_v.at[1 - par], isem)

        gathers = [None] * GRP
        scatters = [None] * GRP
        gathers[0] = pltpu.async_copy(y_hbm.at[src_v.at[par, 0]],
                                      rows.at[0], gsem)
        for c in range(GRP):
            gathers[c].wait()
            scatters[c] = pltpu.async_copy(rows.at[c % 2],
                                           acc.at[dst_v.at[par, c]],
                                           ssem, add=True)
            if c + 1 < GRP:
                if c >= 1:
                    scatters[c - 1].wait()
                gathers[c + 1] = pltpu.async_copy(
                    y_hbm.at[src_v.at[par, c + 1]],
                    rows.at[(c + 1) % 2], gsem)
        scatters[GRP - 2].wait()
        scatters[GRP - 1].wait()
        return carry

    lax.fori_loop(0, AGG_NGRP, group, 0)

    plsc.subcore_barrier()

    pltpu.sync_copy(acc.at[pl.ds(row0, RPS)], out_hbm.at[pl.ds(row0, RPS)])

    @pl.when(s == NS - 1)
    def _():
        pltpu.sync_copy(acc.at[pl.ds(NS * RPS, REM)],
                        out_hbm.at[pl.ds(NS * RPS, REM)])


def _tc_first_body(deg_ref, x_ref, w_ref, y_ref, d_ref):
    deg = jnp.sum(deg_ref[...], axis=1) + 1.0
    dis = lax.rsqrt(deg)[:, None]
    d_ref[...] = dis
    y_ref[...] = jnp.dot(x_ref[...], w_ref[...],
                         preferred_element_type=jnp.float32) * dis


def _tc_mid_body(agg_ref, d_ref, b_ref, w_ref, y_ref):
    dis = d_ref[...]
    x = jax.nn.gelu(agg_ref[...] * dis + b_ref[...])
    y_ref[...] = jnp.dot(x, w_ref[...],
                         preferred_element_type=jnp.float32) * dis


def _tc_pool_body(agg_ref, d_ref, b_ref, batch_ref, out_ref, sums, counts):
    i = pl.program_id(0)

    @pl.when(i == 0)
    def _():
        sums[...] = jnp.zeros_like(sums)
        counts[...] = jnp.zeros_like(counts)

    x = jax.nn.gelu(agg_ref[...] * d_ref[...] + b_ref[...])
    onehot = (batch_ref[...] ==
              lax.broadcasted_iota(jnp.int32, (BLK, G), 1)
              ).astype(jnp.float32)
    sums[...] += lax.dot_general(onehot, x, (((0,), (0,)), ((), ())),
                                 preferred_element_type=jnp.float32)
    counts[...] += jnp.sum(onehot, axis=0)[:, None]

    @pl.when(i == pl.num_programs(0) - 1)
    def _():
        out_ref[...] = sums[...] / jnp.maximum(counts[...], 1.0)


_tc_first = pl.pallas_call(
    _tc_first_body,
    grid=(GRID,),
    in_specs=[
        pl.BlockSpec((BLK, NW), lambda i: (i, 0)),
        pl.BlockSpec((BLK, D), lambda i: (i, 0)),
        pl.BlockSpec((D, D), lambda i: (0, 0)),
    ],
    out_specs=[
        pl.BlockSpec((BLK, D), lambda i: (i, 0)),
        pl.BlockSpec((BLK, 1), lambda i: (i, 0)),
    ],
    out_shape=[
        jax.ShapeDtypeStruct((N, D), jnp.float32),
        jax.ShapeDtypeStruct((N, 1), jnp.float32),
    ],
)

_tc_mid = pl.pallas_call(
    _tc_mid_body,
    grid=(GRID,),
    in_specs=[
        pl.BlockSpec((BLK, D), lambda i: (i, 0)),
        pl.BlockSpec((BLK, 1), lambda i: (i, 0)),
        pl.BlockSpec((1, D), lambda i: (0, 0)),
        pl.BlockSpec((D, D), lambda i: (0, 0)),
    ],
    out_specs=pl.BlockSpec((BLK, D), lambda i: (i, 0)),
    out_shape=jax.ShapeDtypeStruct((N, D), jnp.float32),
)

_tc_pool = pl.pallas_call(
    _tc_pool_body,
    grid=(GRID,),
    in_specs=[
        pl.BlockSpec((BLK, D), lambda i: (i, 0)),
        pl.BlockSpec((BLK, 1), lambda i: (i, 0)),
        pl.BlockSpec((1, D), lambda i: (0, 0)),
        pl.BlockSpec((BLK, 1), lambda i: (i, 0)),
    ],
    out_specs=pl.BlockSpec((G, D), lambda i: (0, 0)),
    out_shape=jax.ShapeDtypeStruct((G, D), jnp.float32),
    scratch_shapes=[
        pltpu.VMEM((G, D), jnp.float32),
        pltpu.VMEM((G, 1), jnp.float32),
    ],
)


def kernel(region_features, region_edges, region_features_batch,
           W1, b1, W2, b2, W3, b3):
    pad_src = (jnp.arange(EPAD, dtype=jnp.int32) * 13) % N
    pad_dst = N + (jnp.arange(EPAD, dtype=jnp.int32) % NDUMP)
    src = jnp.concatenate([region_edges[0], pad_src]).reshape(
        NS * AGG_NCH, CHUNK)
    dst_flat = region_edges[1]
    dst = jnp.concatenate([dst_flat, pad_dst]).reshape(NS * AGG_NCH, CHUNK)
    batch = region_features_batch.reshape(N, 1)

    deg_part = _deg_kernel(dst_flat)
    y1, d = _tc_first(deg_part.T, region_features, W1)
    agg1 = _agg_kernel(y1, src, dst)
    y2 = _tc_mid(agg1, d, b1.reshape(1, D), W2)
    agg2 = _agg_kernel(y2, src, dst)
    y3 = _tc_mid(agg2, d, b2.reshape(1, D), W3)
    agg3 = _agg_kernel(y3, src, dst)
    return _tc_pool(agg3, d, b3.reshape(1, D), batch)


# speedup vs baseline: 24.4339x; 1.6656x over previous
import functools

import jax
import jax.numpy as jnp
from jax import lax
from jax.experimental import pallas as pl
from jax.experimental.pallas import tpu as pltpu
from jax.experimental.pallas import tpu_sc as plsc

N = 10000
E = 320000
D = 128
G = 64
NC = 2
NS = 16
NW = NC * NS
CHUNK = 128
GRP = 8
RPS = 624
REM = N - NS * RPS
NDUMP = 8
NP = N + NDUMP
BLK = 1000
GRID = N // BLK

_sc_mesh2 = plsc.VectorSubcoreMesh(core_axis_name="c", subcore_axis_name="s")

DEG_EPW = E // NW
AGG_NCH = 80
AGG_NGRP = AGG_NCH // GRP
EPAD = NW * AGG_NCH * CHUNK - E


@functools.partial(
    pl.kernel,
    mesh=_sc_mesh2,
    out_type=jax.ShapeDtypeStruct((NW, N), jnp.float32),
    scratch_types=[
        pltpu.VMEM((N,), jnp.float32),
        pltpu.VMEM((DEG_EPW,), jnp.int32),
    ],
    compiler_params=pltpu.CompilerParams(needs_layout_passes=False),
)
def _deg_kernel(dst_hbm, out_hbm, hist, dst_v):
    c = lax.axis_index("c")
    s = lax.axis_index("s")
    wid = s * NC + c

    zero16 = jnp.zeros((16,), jnp.float32)

    def zbody(i, carry):
        hist[pl.ds(i * 16, 16)] = zero16
        return carry

    lax.fori_loop(0, N // 16, zbody, 0)

    pltpu.sync_copy(dst_hbm.at[pl.ds(wid * DEG_EPW, DEG_EPW)], dst_v)

    ones16 = jnp.ones((16,), jnp.float32)

    def body(i, carry):
        idx = dst_v[pl.ds(i * 16, 16)]
        plsc.addupdate_scatter(hist, [idx], ones16)
        return carry

    lax.fori_loop(0, DEG_EPW // 16, body, 0)

    pltpu.sync_copy(hist, out_hbm.at[wid])


@functools.partial(
    pl.kernel,
    mesh=_sc_mesh2,
    out_type=jax.ShapeDtypeStruct((NC, NP, D), jnp.float32),
    scratch_types=[
        pltpu.VMEM((2, GRP, CHUNK), jnp.int32),
        pltpu.VMEM((2, GRP, CHUNK), jnp.int32),
        pltpu.VMEM((2, CHUNK, D), jnp.float32),
        pltpu.VMEM_SHARED((NP, D), jnp.float32),
        pltpu.SemaphoreType.DMA,
        pltpu.SemaphoreType.DMA,
        pltpu.SemaphoreType.DMA,
    ],
    compiler_params=pltpu.CompilerParams(needs_layout_passes=False),
)
def _agg_kernel(y_hbm, zeros_hbm, src_hbm, dst_hbm, out_hbm,
                src_v, dst_v, rows, acc, gsem, ssem, isem):
    c = lax.axis_index("c")
    s = lax.axis_index("s")
    wid = s * NC + c

    row0 = s * RPS

    @pl.when(c == 0)
    def _():
        pltpu.sync_copy(y_hbm.at[pl.ds(row0, RPS)], acc.at[pl.ds(row0, RPS)])

        @pl.when(s == NS - 1)
        def _():
            pltpu.sync_copy(y_hbm.at[pl.ds(NS * RPS, REM)],
                            acc.at[pl.ds(NS * RPS, REM)])

    @pl.when(c != 0)
    def _():
        pltpu.sync_copy(zeros_hbm.at[pl.ds(row0, RPS)],
                        acc.at[pl.ds(row0, RPS)])

        @pl.when(s == NS - 1)
        def _():
            pltpu.sync_copy(zeros_hbm.at[pl.ds(NS * RPS, REM)],
                            acc.at[pl.ds(NS * RPS, REM)])

    base0 = wid * AGG_NCH
    pltpu.async_copy(src_hbm.at[pl.ds(base0, GRP)], src_v.at[0], isem)
    pltpu.async_copy(dst_hbm.at[pl.ds(base0, GRP)], dst_v.at[0], isem)

    plsc.subcore_barrier()

    def group(g, carry):
        par = g % 2
        base = base0 + g * GRP
        pltpu.make_async_copy(src_hbm.at[pl.ds(base, GRP)],
                              src_v.at[par], isem).wait()
        pltpu.make_async_copy(dst_hbm.at[pl.ds(base, GRP)],
                              dst_v.at[par], isem).wait()

        @pl.when(g + 1 < AGG_NGRP)
        def _():
            nbase = base + GRP
            pltpu.async_copy(src_hbm.at[pl.ds(nbase, GRP)],
                             src_v.at[1 - par], isem)
            pltpu.async_copy(dst_hbm.at[pl.ds(nbase, GRP)],
                             dst_v.at[1 - par], isem)

        gathers = [None] * GRP
        scatters = [None] * GRP
        gathers[0] = pltpu.async_copy(y_hbm.at[src_v.at[par, 0]],
                                      rows.at[0], gsem)
        for c in range(GRP):
            gathers[c].wait()
            scatters[c] = pltpu.async_copy(rows.at[c % 2],
                                           acc.at[dst_v.at[par, c]],
                                           ssem, add=True)
            if c + 1 < GRP:
                if c >= 1:
                    scatters[c - 1].wait()
                gathers[c + 1] = pltpu.async_copy(
                    y_hbm.at[src_v.at[par, c + 1]],
                    rows.at[(c + 1) % 2], gsem)
        scatters[GRP - 2].wait()
        scatters[GRP - 1].wait()
        return carry

    lax.fori_loop(0, AGG_NGRP, group, 0)

    plsc.subcore_barrier()

    pltpu.sync_copy(acc.at[pl.ds(row0, RPS)],
                    out_hbm.at[c, pl.ds(row0, RPS)])

    @pl.when(s == NS - 1)
    def _():
        pltpu.sync_copy(acc.at[pl.ds(NS * RPS, REM)],
                        out_hbm.at[c, pl.ds(NS * RPS, REM)])


def _tc_first_body(deg_ref, x_ref, w_ref, y_ref, d_ref):
    deg = jnp.sum(deg_ref[...], axis=1) + 1.0
    dis = lax.rsqrt(deg)[:, None]
    d_ref[...] = dis
    y_ref[...] = jnp.dot(x_ref[...], w_ref[...],
                         preferred_element_type=jnp.float32) * dis


def _tc_mid_body(agg_ref, d_ref, b_ref, w_ref, y_ref):
    dis = d_ref[...]
    x = jax.nn.gelu((agg_ref[0] + agg_ref[1]) * dis + b_ref[...])
    y_ref[...] = jnp.dot(x, w_ref[...],
                         preferred_element_type=jnp.float32) * dis


def _tc_pool_body(agg_ref, d_ref, b_ref, batch_ref, out_ref, sums, counts):
    i = pl.program_id(0)

    @pl.when(i == 0)
    def _():
        sums[...] = jnp.zeros_like(sums)
        counts[...] = jnp.zeros_like(counts)

    x = jax.nn.gelu((agg_ref[0] + agg_ref[1]) * d_ref[...] + b_ref[...])
    onehot = (batch_ref[...] ==
              lax.broadcasted_iota(jnp.int32, (BLK, G), 1)
              ).astype(jnp.float32)
    sums[...] += lax.dot_general(onehot, x, (((0,), (0,)), ((), ())),
                                 preferred_element_type=jnp.float32)
    counts[...] += jnp.sum(onehot, axis=0)[:, None]

    @pl.when(i == pl.num_programs(0) - 1)
    def _():
        out_ref[...] = sums[...] / jnp.maximum(counts[...], 1.0)


_tc_first = pl.pallas_call(
    _tc_first_body,
    grid=(GRID,),
    in_specs=[
        pl.BlockSpec((BLK, NW), lambda i: (i, 0)),
        pl.BlockSpec((BLK, D), lambda i: (i, 0)),
        pl.BlockSpec((D, D), lambda i: (0, 0)),
    ],
    out_specs=[
        pl.BlockSpec((BLK, D), lambda i: (i, 0)),
        pl.BlockSpec((BLK, 1), lambda i: (i, 0)),
    ],
    out_shape=[
        jax.ShapeDtypeStruct((N, D), jnp.float32),
        jax.ShapeDtypeStruct((N, 1), jnp.float32),
    ],
)

_tc_mid = pl.pallas_call(
    _tc_mid_body,
    grid=(GRID,),
    in_specs=[
        pl.BlockSpec((NC, BLK, D), lambda i: (0, i, 0)),
        pl.BlockSpec((BLK, 1), lambda i: (i, 0)),
        pl.BlockSpec((1, D), lambda i: (0, 0)),
        pl.BlockSpec((D, D), lambda i: (0, 0)),
    ],
    out_specs=pl.BlockSpec((BLK, D), lambda i: (i, 0)),
    out_shape=jax.ShapeDtypeStruct((N, D), jnp.float32),
)

_tc_pool = pl.pallas_call(
    _tc_pool_body,
    grid=(GRID,),
    in_specs=[
        pl.BlockSpec((NC, BLK, D), lambda i: (0, i, 0)),
        pl.BlockSpec((BLK, 1), lambda i: (i, 0)),
        pl.BlockSpec((1, D), lambda i: (0, 0)),
        pl.BlockSpec((BLK, 1), lambda i: (i, 0)),
    ],
    out_specs=pl.BlockSpec((G, D), lambda i: (0, 0)),
    out_shape=jax.ShapeDtypeStruct((G, D), jnp.float32),
    scratch_shapes=[
        pltpu.VMEM((G, D), jnp.float32),
        pltpu.VMEM((G, 1), jnp.float32),
    ],
)


def kernel(region_features, region_edges, region_features_batch,
           W1, b1, W2, b2, W3, b3):
    pad_src = (jnp.arange(EPAD, dtype=jnp.int32) * 13) % N
    pad_dst = N + (jnp.arange(EPAD, dtype=jnp.int32) % NDUMP)
    src = jnp.concatenate([region_edges[0], pad_src]).reshape(
        NW * AGG_NCH, CHUNK)
    dst_flat = region_edges[1]
    dst = jnp.concatenate([dst_flat, pad_dst]).reshape(NW * AGG_NCH, CHUNK)
    batch = region_features_batch.reshape(N, 1)
    zeros = jnp.zeros((NP, D), jnp.float32)

    deg_part = _deg_kernel(dst_flat)
    y1, d = _tc_first(deg_part.T, region_features, W1)
    agg1 = _agg_kernel(y1, zeros, src, dst)
    y2 = _tc_mid(agg1, d, b1.reshape(1, D), W2)
    agg2 = _agg_kernel(y2, zeros, src, dst)
    y3 = _tc_mid(agg2, d, b2.reshape(1, D), W3)
    agg3 = _agg_kernel(y3, zeros, src, dst)
    return _tc_pool(agg3, d, b3.reshape(1, D), batch)


# speedup vs baseline: 25.1203x; 1.0281x over previous
import functools

import jax
import jax.numpy as jnp
from jax import lax
from jax.experimental import pallas as pl
from jax.experimental.pallas import tpu as pltpu
from jax.experimental.pallas import tpu_sc as plsc

N = 10000
E = 320000
D = 128
G = 64
NC = 2
NS = 16
NW = NC * NS
CHUNK = 128
GRP = 8
RPS = 624
REM = N - NS * RPS
NDUMP = 8
NP = N + NDUMP
BLK = 1000
GRID = N // BLK

_sc_mesh2 = plsc.VectorSubcoreMesh(core_axis_name="c", subcore_axis_name="s")

DEG_EPW = E // NW
AGG_NCH = 80
AGG_NGRP = AGG_NCH // GRP
EPAD = NW * AGG_NCH * CHUNK - E


@functools.partial(
    pl.kernel,
    mesh=_sc_mesh2,
    out_type=jax.ShapeDtypeStruct((NW, N), jnp.float32),
    scratch_types=[
        pltpu.VMEM((N,), jnp.float32),
        pltpu.VMEM((DEG_EPW,), jnp.int32),
    ],
    compiler_params=pltpu.CompilerParams(needs_layout_passes=False),
)
def _deg_kernel(dst_hbm, out_hbm, hist, dst_v):
    c = lax.axis_index("c")
    s = lax.axis_index("s")
    wid = s * NC + c

    zero16 = jnp.zeros((16,), jnp.float32)

    def zbody(i, carry):
        hist[pl.ds(i * 16, 16)] = zero16
        return carry

    lax.fori_loop(0, N // 16, zbody, 0)

    pltpu.sync_copy(dst_hbm.at[pl.ds(wid * DEG_EPW, DEG_EPW)], dst_v)

    ones16 = jnp.ones((16,), jnp.float32)

    def body(i, carry):
        idx = dst_v[pl.ds(i * 16, 16)]
        plsc.addupdate_scatter(hist, [idx], ones16)
        return carry

    lax.fori_loop(0, DEG_EPW // 16, body, 0)

    pltpu.sync_copy(hist, out_hbm.at[wid])


@functools.partial(
    pl.kernel,
    mesh=_sc_mesh2,
    out_type=jax.ShapeDtypeStruct((NC, NP, D), jnp.float32),
    scratch_types=[
        pltpu.VMEM((2, GRP, CHUNK), jnp.int32),
        pltpu.VMEM((2, GRP, CHUNK), jnp.int32),
        pltpu.VMEM((2, CHUNK, D), jnp.float32),
        pltpu.VMEM_SHARED((NP, D), jnp.float32),
        pltpu.SemaphoreType.DMA,
        pltpu.SemaphoreType.DMA,
        pltpu.SemaphoreType.DMA,
    ],
    compiler_params=pltpu.CompilerParams(needs_layout_passes=False),
)
def _agg_kernel(y_hbm, zeros_hbm, src_hbm, dst_hbm, out_hbm,
                src_v, dst_v, rows, acc, gsem, ssem, isem):
    c = lax.axis_index("c")
    s = lax.axis_index("s")
    wid = s * NC + c

    row0 = s * RPS

    @pl.when(c == 0)
    def _():
        pltpu.sync_copy(y_hbm.at[pl.ds(row0, RPS)], acc.at[pl.ds(row0, RPS)])

        @pl.when(s == NS - 1)
        def _():
            pltpu.sync_copy(y_hbm.at[pl.ds(NS * RPS, REM)],
                            acc.at[pl.ds(NS * RPS, REM)])

    @pl.when(c != 0)
    def _():
        pltpu.sync_copy(zeros_hbm.at[pl.ds(row0, RPS)],
                        acc.at[pl.ds(row0, RPS)])

        @pl.when(s == NS - 1)
        def _():
            pltpu.sync_copy(zeros_hbm.at[pl.ds(NS * RPS, REM)],
                            acc.at[pl.ds(NS * RPS, REM)])

    base0 = wid * AGG_NCH
    pltpu.async_copy(src_hbm.at[pl.ds(base0, GRP)], src_v.at[0], isem)
    pltpu.async_copy(dst_hbm.at[pl.ds(base0, GRP)], dst_v.at[0], isem)
    pltpu.async_copy(src_hbm.at[pl.ds(base0 + GRP, GRP)], src_v.at[1], isem)
    pltpu.async_copy(dst_hbm.at[pl.ds(base0 + GRP, GRP)], dst_v.at[1], isem)

    plsc.subcore_barrier()

    pltpu.make_async_copy(src_hbm.at[pl.ds(base0, GRP)],
                          src_v.at[0], isem).wait()
    pltpu.make_async_copy(dst_hbm.at[pl.ds(base0, GRP)],
                          dst_v.at[0], isem).wait()
    pltpu.async_copy(y_hbm.at[src_v.at[0, 0]], rows.at[0], gsem)

    def chunk(j, carry):
        par = j % 2
        gpar = (j // GRP) % 2
        row = j % GRP
        pltpu.make_async_copy(y_hbm.at[src_v.at[gpar, row]],
                              rows.at[par], gsem).wait()

        @pl.when(j >= 1)
        def _():
            pltpu.make_async_copy(rows.at[1 - par],
                                  acc.at[dst_v.at[gpar, row]], ssem).wait()

        pltpu.async_copy(rows.at[par], acc.at[dst_v.at[gpar, row]],
                         ssem, add=True)

        nj = j + 1
        ng = nj // GRP
        ngpar = ng % 2

        @pl.when(nj < AGG_NCH)
        def _():
            @pl.when(nj % GRP == 0)
            def _():
                nbase = base0 + ng * GRP
                pltpu.make_async_copy(src_hbm.at[pl.ds(nbase, GRP)],
                                      src_v.at[ngpar], isem).wait()
                pltpu.make_async_copy(dst_hbm.at[pl.ds(nbase, GRP)],
                                      dst_v.at[ngpar], isem).wait()

                @pl.when(ng + 1 < AGG_NGRP)
                def _():
                    fbase = base0 + (ng + 1) * GRP
                    pltpu.async_copy(src_hbm.at[pl.ds(fbase, GRP)],
                                     src_v.at[1 - ngpar], isem)
                    pltpu.async_copy(dst_hbm.at[pl.ds(fbase, GRP)],
                                     dst_v.at[1 - ngpar], isem)

            pltpu.async_copy(y_hbm.at[src_v.at[ngpar, nj % GRP]],
                             rows.at[nj % 2], gsem)
        return carry

    lax.fori_loop(0, AGG_NCH, chunk, 0)

    pltpu.make_async_copy(rows.at[(AGG_NCH - 1) % 2],
                          acc.at[dst_v.at[(AGG_NGRP - 1) % 2, GRP - 1]],
                          ssem).wait()

    plsc.subcore_barrier()

    pltpu.sync_copy(acc.at[pl.ds(row0, RPS)],
                    out_hbm.at[c, pl.ds(row0, RPS)])

    @pl.when(s == NS - 1)
    def _():
        pltpu.sync_copy(acc.at[pl.ds(NS * RPS, REM)],
                        out_hbm.at[c, pl.ds(NS * RPS, REM)])


def _tc_first_body(deg_ref, x_ref, w_ref, y_ref, d_ref):
    deg = jnp.sum(deg_ref[...], axis=1) + 1.0
    dis = lax.rsqrt(deg)[:, None]
    d_ref[...] = dis
    y_ref[...] = jnp.dot(x_ref[...], w_ref[...],
                         preferred_element_type=jnp.float32) * dis


def _tc_mid_body(agg_ref, d_ref, b_ref, w_ref, y_ref):
    dis = d_ref[...]
    x = jax.nn.gelu((agg_ref[0] + agg_ref[1]) * dis + b_ref[...])
    y_ref[...] = jnp.dot(x, w_ref[...],
                         preferred_element_type=jnp.float32) * dis


def _tc_pool_body(agg_ref, d_ref, b_ref, batch_ref, out_ref, sums, counts):
    i = pl.program_id(0)

    @pl.when(i == 0)
    def _():
        sums[...] = jnp.zeros_like(sums)
        counts[...] = jnp.zeros_like(counts)

    x = jax.nn.gelu((agg_ref[0] + agg_ref[1]) * d_ref[...] + b_ref[...])
    onehot = (batch_ref[...] ==
              lax.broadcasted_iota(jnp.int32, (BLK, G), 1)
              ).astype(jnp.float32)
    sums[...] += lax.dot_general(onehot, x, (((0,), (0,)), ((), ())),
                                 preferred_element_type=jnp.float32)
    counts[...] += jnp.sum(onehot, axis=0)[:, None]

    @pl.when(i == pl.num_programs(0) - 1)
    def _():
        out_ref[...] = sums[...] / jnp.maximum(counts[...], 1.0)


_tc_first = pl.pallas_call(
    _tc_first_body,
    grid=(GRID,),
    in_specs=[
        pl.BlockSpec((BLK, NW), lambda i: (i, 0)),
        pl.BlockSpec((BLK, D), lambda i: (i, 0)),
        pl.BlockSpec((D, D), lambda i: (0, 0)),
    ],
    out_specs=[
        pl.BlockSpec((BLK, D), lambda i: (i, 0)),
        pl.BlockSpec((BLK, 1), lambda i: (i, 0)),
    ],
    out_shape=[
        jax.ShapeDtypeStruct((N, D), jnp.float32),
        jax.ShapeDtypeStruct((N, 1), jnp.float32),
    ],
)

_tc_mid = pl.pallas_call(
    _tc_mid_body,
    grid=(GRID,),
    in_specs=[
        pl.BlockSpec((NC, BLK, D), lambda i: (0, i, 0)),
        pl.BlockSpec((BLK, 1), lambda i: (i, 0)),
        pl.BlockSpec((1, D), lambda i: (0, 0)),
        pl.BlockSpec((D, D), lambda i: (0, 0)),
    ],
    out_specs=pl.BlockSpec((BLK, D), lambda i: (i, 0)),
    out_shape=jax.ShapeDtypeStruct((N, D), jnp.float32),
)

_tc_pool = pl.pallas_call(
    _tc_pool_body,
    grid=(GRID,),
    in_specs=[
        pl.BlockSpec((NC, BLK, D), lambda i: (0, i, 0)),
        pl.BlockSpec((BLK, 1), lambda i: (i, 0)),
        pl.BlockSpec((1, D), lambda i: (0, 0)),
        pl.BlockSpec((BLK, 1), lambda i: (i, 0)),
    ],
    out_specs=pl.BlockSpec((G, D), lambda i: (0, 0)),
    out_shape=jax.ShapeDtypeStruct((G, D), jnp.float32),
    scratch_shapes=[
        pltpu.VMEM((G, D), jnp.float32),
        pltpu.VMEM((G, 1), jnp.float32),
    ],
)


def kernel(region_features, region_edges, region_features_batch,
           W1, b1, W2, b2, W3, b3):
    pad_src = (jnp.arange(EPAD, dtype=jnp.int32) * 13) % N
    pad_dst = N + (jnp.arange(EPAD, dtype=jnp.int32) % NDUMP)
    src = jnp.concatenate([region_edges[0], pad_src]).reshape(
        NW * AGG_NCH, CHUNK)
    dst_flat = region_edges[1]
    dst = jnp.concatenate([dst_flat, pad_dst]).reshape(NW * AGG_NCH, CHUNK)
    batch = region_features_batch.reshape(N, 1)
    zeros = jnp.zeros((NP, D), jnp.float32)

    deg_part = _deg_kernel(dst_flat)
    y1, d = _tc_first(deg_part.T, region_features, W1)
    agg1 = _agg_kernel(y1, zeros, src, dst)
    y2 = _tc_mid(agg1, d, b1.reshape(1, D), W2)
    agg2 = _agg_kernel(y2, zeros, src, dst)
    y3 = _tc_mid(agg2, d, b2.reshape(1, D), W3)
    agg3 = _agg_kernel(y3, zeros, src, dst)
    return _tc_pool(agg3, d, b3.reshape(1, D), batch)


# speedup vs baseline: 25.3656x; 1.0098x over previous
import functools

import jax
import jax.numpy as jnp
from jax import lax
from jax.experimental import pallas as pl
from jax.experimental.pallas import tpu as pltpu
from jax.experimental.pallas import tpu_sc as plsc

N = 10000
E = 320000
D = 128
G = 64
NC = 2
NS = 16
NW = NC * NS
CHUNK = 128
GRP = 8
RPS = 624
REM = N - NS * RPS
NDUMP = 64
NP = N + NDUMP
BLK = 1000
GRID = N // BLK

_sc_mesh2 = plsc.VectorSubcoreMesh(core_axis_name="c", subcore_axis_name="s")

DEG_EPW = E // NW
AGG_NCH = 80
AGG_NGRP = AGG_NCH // GRP
EPAD = NW * AGG_NCH * CHUNK - E


@functools.partial(
    pl.kernel,
    mesh=_sc_mesh2,
    out_type=jax.ShapeDtypeStruct((NW, N), jnp.float32),
    scratch_types=[
        pltpu.VMEM((N,), jnp.float32),
        pltpu.VMEM((DEG_EPW,), jnp.int32),
    ],
    compiler_params=pltpu.CompilerParams(needs_layout_passes=False),
)
def _deg_kernel(dst_hbm, out_hbm, hist, dst_v):
    c = lax.axis_index("c")
    s = lax.axis_index("s")
    wid = s * NC + c

    zero16 = jnp.zeros((16,), jnp.float32)

    def zbody(i, carry):
        hist[pl.ds(i * 16, 16)] = zero16
        return carry

    lax.fori_loop(0, N // 16, zbody, 0)

    pltpu.sync_copy(dst_hbm.at[pl.ds(wid * DEG_EPW, DEG_EPW)], dst_v)

    ones16 = jnp.ones((16,), jnp.float32)

    def body(i, carry):
        idx = dst_v[pl.ds(i * 16, 16)]
        plsc.addupdate_scatter(hist, [idx], ones16)
        return carry

    lax.fori_loop(0, DEG_EPW // 16, body, 0)

    pltpu.sync_copy(hist, out_hbm.at[wid])


@functools.partial(
    pl.kernel,
    mesh=_sc_mesh2,
    out_type=jax.ShapeDtypeStruct((NC, NP, D), jnp.float32),
    scratch_types=[
        pltpu.VMEM((2, GRP, CHUNK), jnp.int32),
        pltpu.VMEM((2, GRP, CHUNK), jnp.int32),
        pltpu.VMEM((2, CHUNK, D), jnp.float32),
        pltpu.VMEM_SHARED((NP, D), jnp.float32),
        pltpu.SemaphoreType.DMA,
        pltpu.SemaphoreType.DMA,
        pltpu.SemaphoreType.DMA,
        pltpu.SemaphoreType.DMA,
    ],
    compiler_params=pltpu.CompilerParams(needs_layout_passes=False),
)
def _agg_kernel(y_hbm, src_hbm, dst_hbm, out_hbm,
                src_v, dst_v, rows, acc, gsem, ssem0, ssem1, isem):
    c = lax.axis_index("c")
    s = lax.axis_index("s")
    wid = s * NC + c
    row0 = s * RPS
    base0 = wid * AGG_NCH

    pltpu.async_copy(src_hbm.at[pl.ds(base0, GRP)], src_v.at[0], isem)
    pltpu.async_copy(dst_hbm.at[pl.ds(base0, GRP)], dst_v.at[0], isem)
    pltpu.async_copy(src_hbm.at[pl.ds(base0 + GRP, GRP)], src_v.at[1], isem)
    pltpu.async_copy(dst_hbm.at[pl.ds(base0 + GRP, GRP)], dst_v.at[1], isem)

    @pl.when(c == 0)
    def _():
        pltpu.sync_copy(y_hbm.at[pl.ds(row0, RPS)], acc.at[pl.ds(row0, RPS)])

        @pl.when(s == NS - 1)
        def _():
            pltpu.sync_copy(y_hbm.at[pl.ds(NS * RPS, REM)],
                            acc.at[pl.ds(NS * RPS, REM)])

    @pl.when(c != 0)
    def _():
        zero16 = jnp.zeros((16,), jnp.float32)

        def zrow(r, carry):
            for k in range(D // 16):
                rows[0, r, pl.ds(k * 16, 16)] = zero16
            return carry

        lax.fori_loop(0, CHUNK, zrow, 0)
        for t in range(RPS // CHUNK):
            pltpu.sync_copy(rows.at[0],
                            acc.at[pl.ds(row0 + t * CHUNK, CHUNK)])
        tail = RPS - (RPS // CHUNK) * CHUNK
        pltpu.sync_copy(rows.at[0, pl.ds(0, tail)],
                        acc.at[pl.ds(row0 + RPS - tail, tail)])

        @pl.when(s == NS - 1)
        def _():
            pltpu.sync_copy(rows.at[0, pl.ds(0, REM)],
                            acc.at[pl.ds(NS * RPS, REM)])

    plsc.subcore_barrier()

    pltpu.make_async_copy(src_hbm.at[pl.ds(base0, GRP)],
                          src_v.at[0], isem).wait()
    pltpu.make_async_copy(dst_hbm.at[pl.ds(base0, GRP)],
                          dst_v.at[0], isem).wait()
    pltpu.async_copy(y_hbm.at[src_v.at[0, 0]], rows.at[0], gsem)

    ssems = (ssem0, ssem1)

    def group(g, carry):
        gpar = g % 2
        for cc in range(GRP):
            par = cc % 2
            pltpu.make_async_copy(y_hbm.at[src_v.at[gpar, cc]],
                                  rows.at[par], gsem).wait()
            pltpu.async_copy(rows.at[par], acc.at[dst_v.at[gpar, cc]],
                             ssems[par], add=True)
            if cc == 0:
                @pl.when(g >= 1)
                def _():
                    pltpu.make_async_copy(rows.at[1 - par],
                                          acc.at[dst_v.at[gpar, cc]],
                                          ssems[1 - par]).wait()
            else:
                pltpu.make_async_copy(rows.at[1 - par],
                                      acc.at[dst_v.at[gpar, cc]],
                                      ssems[1 - par]).wait()
            if cc + 1 < GRP:
                pltpu.async_copy(y_hbm.at[src_v.at[gpar, cc + 1]],
                                 rows.at[1 - par], gsem)
            else:
                @pl.when(g + 1 < AGG_NGRP)
                def _():
                    ngpar = 1 - gpar
                    nbase = base0 + (g + 1) * GRP
                    pltpu.make_async_copy(src_hbm.at[pl.ds(nbase, GRP)],
                                          src_v.at[ngpar], isem).wait()
                    pltpu.make_async_copy(dst_hbm.at[pl.ds(nbase, GRP)],
                                          dst_v.at[ngpar], isem).wait()

                    @pl.when(g + 2 < AGG_NGRP)
                    def _():
                        fbase = base0 + (g + 2) * GRP
                        pltpu.async_copy(src_hbm.at[pl.ds(fbase, GRP)],
                                         src_v.at[1 - ngpar], isem)
                        pltpu.async_copy(dst_hbm.at[pl.ds(fbase, GRP)],
                                         dst_v.at[1 - ngpar], isem)

                    pltpu.async_copy(y_hbm.at[src_v.at[ngpar, 0]],
                                     rows.at[1 - par], gsem)
        return carry

    lax.fori_loop(0, AGG_NGRP, group, 0)

    lastpar = (GRP - 1) % 2
    pltpu.make_async_copy(rows.at[lastpar],
                          acc.at[dst_v.at[(AGG_NGRP - 1) % 2, GRP - 1]],
                          ssems[lastpar]).wait()

    plsc.subcore_barrier()

    pltpu.sync_copy(acc.at[pl.ds(row0, RPS)],
                    out_hbm.at[c, pl.ds(row0, RPS)])

    @pl.when(s == NS - 1)
    def _():
        pltpu.sync_copy(acc.at[pl.ds(NS * RPS, REM)],
                        out_hbm.at[c, pl.ds(NS * RPS, REM)])


def _tc_h1_body(x_ref, w_ref, h_ref):
    h_ref[...] = jnp.dot(x_ref[...], w_ref[...],
                         preferred_element_type=jnp.float32)


def _tc_first_body(deg_ref, h_ref, y_ref, d_ref):
    deg = jnp.sum(deg_ref[...], axis=1) + 1.0
    dis = lax.rsqrt(deg)[:, None]
    d_ref[...] = dis
    y_ref[...] = h_ref[...] * dis


def _tc_mid_body(agg_ref, d_ref, b_ref, w_ref, y_ref):
    dis = d_ref[...]
    x = jax.nn.gelu((agg_ref[0] + agg_ref[1]) * dis + b_ref[...])
    y_ref[...] = jnp.dot(x, w_ref[...],
                         preferred_element_type=jnp.float32) * dis


def _tc_pool_body(agg_ref, d_ref, b_ref, batch_ref, out_ref, sums, counts):
    i = pl.program_id(0)

    @pl.when(i == 0)
    def _():
        sums[...] = jnp.zeros_like(sums)
        counts[...] = jnp.zeros_like(counts)

    x = jax.nn.gelu((agg_ref[0] + agg_ref[1]) * d_ref[...] + b_ref[...])
    onehot = (batch_ref[...] ==
              lax.broadcasted_iota(jnp.int32, (BLK, G), 1)
              ).astype(jnp.float32)
    sums[...] += lax.dot_general(onehot, x, (((0,), (0,)), ((), ())),
                                 preferred_element_type=jnp.float32)
    counts[...] += jnp.sum(onehot, axis=0)[:, None]

    @pl.when(i == pl.num_programs(0) - 1)
    def _():
        out_ref[...] = sums[...] / jnp.maximum(counts[...], 1.0)


_tc_h1 = pl.pallas_call(
    _tc_h1_body,
    grid=(GRID,),
    in_specs=[
        pl.BlockSpec((BLK, D), lambda i: (i, 0)),
        pl.BlockSpec((D, D), lambda i: (0, 0)),
    ],
    out_specs=pl.BlockSpec((BLK, D), lambda i: (i, 0)),
    out_shape=jax.ShapeDtypeStruct((N, D), jnp.float32),
)

_tc_first = pl.pallas_call(
    _tc_first_body,
    grid=(GRID,),
    in_specs=[
        pl.BlockSpec((BLK, NW), lambda i: (i, 0)),
        pl.BlockSpec((BLK, D), lambda i: (i, 0)),
    ],
    out_specs=[
        pl.BlockSpec((BLK, D), lambda i: (i, 0)),
        pl.BlockSpec((BLK, 1), lambda i: (i, 0)),
    ],
    out_shape=[
        jax.ShapeDtypeStruct((N, D), jnp.float32),
        jax.ShapeDtypeStruct((N, 1), jnp.float32),
    ],
)

_tc_mid = pl.pallas_call(
    _tc_mid_body,
    grid=(GRID,),
    in_specs=[
        pl.BlockSpec((NC, BLK, D), lambda i: (0, i, 0)),
        pl.BlockSpec((BLK, 1), lambda i: (i, 0)),
        pl.BlockSpec((1, D), lambda i: (0, 0)),
        pl.BlockSpec((D, D), lambda i: (0, 0)),
    ],
    out_specs=pl.BlockSpec((BLK, D), lambda i: (i, 0)),
    out_shape=jax.ShapeDtypeStruct((N, D), jnp.float32),
)

_tc_pool = pl.pallas_call(
    _tc_pool_body,
    grid=(GRID,),
    in_specs=[
        pl.BlockSpec((NC, BLK, D), lambda i: (0, i, 0)),
        pl.BlockSpec((BLK, 1), lambda i: (i, 0)),
        pl.BlockSpec((1, D), lambda i: (0, 0)),
        pl.BlockSpec((BLK, 1), lambda i: (i, 0)),
    ],
    out_specs=pl.BlockSpec((G, D), lambda i: (0, 0)),
    out_shape=jax.ShapeDtypeStruct((G, D), jnp.float32),
    scratch_shapes=[
        pltpu.VMEM((G, D), jnp.float32),
        pltpu.VMEM((G, 1), jnp.float32),
    ],
)


def kernel(region_features, region_edges, region_features_batch,
           W1, b1, W2, b2, W3, b3):
    pad_src = (jnp.arange(EPAD, dtype=jnp.int32) * 13) % N
    pad_dst = N + (jnp.arange(EPAD, dtype=jnp.int32) % NDUMP)
    src = jnp.concatenate([region_edges[0], pad_src]).reshape(
        NW * AGG_NCH, CHUNK)
    dst_flat = region_edges[1]
    dst = jnp.concatenate([dst_flat, pad_dst]).reshape(NW * AGG_NCH, CHUNK)
    batch = region_features_batch.reshape(N, 1)

    deg_part = _deg_kernel(dst_flat)
    h1 = _tc_h1(region_features, W1)
    y1, d = _tc_first(deg_part.T, h1)
    agg1 = _agg_kernel(y1, src, dst)
    y2 = _tc_mid(agg1, d, b1.reshape(1, D), W2)
    agg2 = _agg_kernel(y2, src, dst)
    y3 = _tc_mid(agg2, d, b2.reshape(1, D), W3)
    agg3 = _agg_kernel(y3, src, dst)
    return _tc_pool(agg3, d, b3.reshape(1, D), batch)
